# Initial kernel scaffold; baseline (speedup 1.0000x reference)
import jax
import jax.numpy as jnp
from jax.experimental import pallas as pl


def kernel(x, indices):
    raise NotImplementedError("write your pallas kernel here")


# speedup vs baseline: 128.1895x; 128.1895x over previous
import functools

import jax
import jax.numpy as jnp
from jax import lax
from jax.experimental import pallas as pl
from jax.experimental.pallas import tpu as pltpu
from jax.experimental.pallas import tpu_sc as plsc

_B, _C, _H, _W = 4, 96, 128, 128
_KS = 2
_N = _B * _C
_IN = _H * _W
_OUT = (_KS * _H) * (_KS * _W)
_NC, _NS = 2, 16
_NW = _NC * _NS
_PW = _N // _NW
_L = 16


def _unpool_body(x_hbm, idx_hbm, out_hbm, xv, iv, ov):
    wid = lax.axis_index("s") * _NC + lax.axis_index("c")
    zeros = jnp.zeros((_L,), jnp.float32)

    @plsc.parallel_loop(0, _OUT, step=_L, unroll=8)
    def _zero(k):
        ov[pl.ds(k, _L)] = zeros

    @pl.loop(0, _PW)
    def _plane(p):
        plane = wid * _PW + p
        pltpu.sync_copy(x_hbm.at[plane], xv)
        pltpu.sync_copy(idx_hbm.at[plane], iv)

        @plsc.parallel_loop(0, _IN, step=_L, unroll=4)
        def _scat(k):
            plsc.store_scatter(ov, [iv[pl.ds(k, _L)]], xv[pl.ds(k, _L)])

        pltpu.sync_copy(ov, out_hbm.at[plane])

        @plsc.parallel_loop(0, _IN, step=_L, unroll=4)
        def _unscat(k):
            plsc.store_scatter(ov, [iv[pl.ds(k, _L)]], zeros)


_unpool = functools.partial(
    pl.kernel,
    out_type=jax.ShapeDtypeStruct((_N, _OUT), jnp.float32),
    mesh=plsc.VectorSubcoreMesh(
        core_axis_name="c", subcore_axis_name="s",
        num_cores=_NC, num_subcores=_NS,
    ),
    scratch_types=[
        pltpu.VMEM((_IN,), jnp.float32),
        pltpu.VMEM((_IN,), jnp.int32),
        pltpu.VMEM((_OUT,), jnp.float32),
    ],
    compiler_params=pltpu.CompilerParams(needs_layout_passes=False),
)(_unpool_body)


@jax.jit
def kernel(x, indices):
    out = _unpool(x.reshape(_N, _IN), indices.reshape(_N, _IN))
    return out.reshape(_B, _C, _KS * _H, _KS * _W)


# speedup vs baseline: 227.2536x; 1.7728x over previous
import functools

import jax
import jax.numpy as jnp
from jax import lax
from jax.experimental import pallas as pl
from jax.experimental.pallas import tpu as pltpu
from jax.experimental.pallas import tpu_sc as plsc

_B, _C, _H, _W = 4, 96, 128, 128
_KS = 2
_N = _B * _C
_IN = _H * _W
_HO, _WO = _KS * _H, _KS * _W
_NC, _NS = 2, 16
_NW = _NC * _NS
_PW = _N // _NW
_L = 16


def _unpool_body(x_hbm, idx_hbm, out_hbm, xv, iv, ov):
    wid = lax.axis_index("s") * _NC + lax.axis_index("c")
    zeros = jnp.zeros((_L,), jnp.float32)

    @plsc.parallel_loop(0, _HO, step=1, unroll=8)
    def _zero(r):
        @plsc.parallel_loop(0, _WO, step=_L)
        def _zrow(c):
            ov[r, pl.ds(c, _L)] = zeros

    @pl.loop(0, _PW)
    def _plane(p):
        plane = wid * _PW + p
        pltpu.sync_copy(x_hbm.at[plane], xv)
        pltpu.sync_copy(idx_hbm.at[plane], iv)

        @plsc.parallel_loop(0, _IN, step=_L, unroll=4)
        def _scat(k):
            r = lax.shift_right_logical(k, 7)
            c = lax.bitwise_and(k, 127)
            t = iv[r, pl.ds(c, _L)]
            hi = lax.shift_right_logical(t, 8)
            lo = lax.bitwise_and(t, jnp.int32(255))
            plsc.store_scatter(ov, [hi, lo], xv[r, pl.ds(c, _L)])

        pltpu.sync_copy(ov, out_hbm.at[plane])

        @plsc.parallel_loop(0, _IN, step=_L, unroll=4)
        def _unscat(k):
            r = lax.shift_right_logical(k, 7)
            c = lax.bitwise_and(k, 127)
            t = iv[r, pl.ds(c, _L)]
            hi = lax.shift_right_logical(t, 8)
            lo = lax.bitwise_and(t, jnp.int32(255))
            plsc.store_scatter(ov, [hi, lo], zeros)


_unpool = functools.partial(
    pl.kernel,
    out_type=jax.ShapeDtypeStruct((_N, _HO, _WO), jnp.float32),
    mesh=plsc.VectorSubcoreMesh(
        core_axis_name="c", subcore_axis_name="s",
        num_cores=_NC, num_subcores=_NS,
    ),
    scratch_types=[
        pltpu.VMEM((_H, _W), jnp.float32),
        pltpu.VMEM((_H, _W), jnp.int32),
        pltpu.VMEM((_HO, _WO), jnp.float32),
    ],
    compiler_params=pltpu.CompilerParams(
        needs_layout_passes=False, use_tc_tiling_on_sc=True,
    ),
)(_unpool_body)


@jax.jit
def kernel(x, indices):
    out = _unpool(x.reshape(_N, _H, _W), indices.reshape(_N, _H, _W))
    return out.reshape(_B, _C, _HO, _WO)


# speedup vs baseline: 325.6188x; 1.4328x over previous
import functools

import jax
import jax.numpy as jnp
from jax import lax
from jax.experimental import pallas as pl
from jax.experimental.pallas import tpu as pltpu
from jax.experimental.pallas import tpu_sc as plsc

_B, _C, _H, _W = 4, 96, 128, 128
_KS = 2
_N = _B * _C
_HO, _WO = _KS * _H, _KS * _W
_NC, _NS = 2, 16
_NW = _NC * _NS
_PW = _N // _NW
_L = 16
_HH = _H // 2
_HOH = _HO // 2
_CIN = _HH * _W


def _unpool_body(x_hbm, idx_hbm, out_hbm,
                 xv0, xv1, iv0, iv1, tv0, tv1, ov0, ov1,
                 sin0, sin1, sout0, sout1):
    wid = lax.axis_index("s") * _NC + lax.axis_index("c")
    base = wid * _PW
    zeros = jnp.zeros((_L,), jnp.float32)
    xvs, ivs, tvs, ovs = (xv0, xv1), (iv0, iv1), (tv0, tv1), (ov0, ov1)
    sins, souts = (sin0, sin1), (sout0, sout1)

    def start_in(plane, h):
        pltpu.async_copy(x_hbm.at[plane, pl.ds(h * _HH, _HH)], xvs[h], sins[h])
        pltpu.async_copy(idx_hbm.at[plane, pl.ds(h * _HH, _HH)], ivs[h], sins[h])

    def wait_in(plane, h):
        pltpu.make_async_copy(
            x_hbm.at[plane, pl.ds(h * _HH, _HH)], xvs[h], sins[h]).wait()
        pltpu.make_async_copy(
            idx_hbm.at[plane, pl.ds(h * _HH, _HH)], ivs[h], sins[h]).wait()

    def out_slice(plane, h):
        return out_hbm.at[plane, pl.ds(h * _HOH, _HOH)]

    def scatter(h):
        iv, tv, xv, ov = ivs[h], tvs[h], xvs[h], ovs[h]

        @plsc.parallel_loop(0, _CIN, step=_L, unroll=4)
        def _scat(k):
            r = lax.shift_right_logical(k, 7)
            c = lax.bitwise_and(k, 127)
            t = iv[r, pl.ds(c, _L)]
            tv[r, pl.ds(c, _L)] = t
            hi = lax.bitwise_and(lax.shift_right_logical(t, 8), jnp.int32(127))
            lo = lax.bitwise_and(t, jnp.int32(255))
            plsc.store_scatter(ov, [hi, lo], xv[r, pl.ds(c, _L)])

    def unscatter(h):
        tv, ov = tvs[h], ovs[h]

        @plsc.parallel_loop(0, _CIN, step=_L, unroll=4)
        def _unscat(k):
            r = lax.shift_right_logical(k, 7)
            c = lax.bitwise_and(k, 127)
            t = tv[r, pl.ds(c, _L)]
            hi = lax.bitwise_and(lax.shift_right_logical(t, 8), jnp.int32(127))
            lo = lax.bitwise_and(t, jnp.int32(255))
            plsc.store_scatter(ov, [hi, lo], zeros)

    for ov in ovs:
        @plsc.parallel_loop(0, _HOH, step=1, unroll=8)
        def _zero(r):
            @plsc.parallel_loop(0, _WO, step=_L)
            def _zrow(c):
                ov[r, pl.ds(c, _L)] = zeros

    start_in(base, 0)
    start_in(base, 1)

    @pl.loop(0, _PW)
    def _plane(p):
        plane = base + p

        wait_in(plane, 0)
        scatter(0)
        pltpu.async_copy(ov0, out_slice(plane, 0), sout0)

        @pl.when(p < _PW - 1)
        def _prefetch0():
            start_in(plane + 1, 0)

        @pl.when(p > 0)
        def _drain1():
            pltpu.make_async_copy(ov1, out_slice(plane - 1, 1), sout1).wait()
            unscatter(1)

        wait_in(plane, 1)
        scatter(1)
        pltpu.async_copy(ov1, out_slice(plane, 1), sout1)

        @pl.when(p < _PW - 1)
        def _prefetch1():
            start_in(plane + 1, 1)

        pltpu.make_async_copy(ov0, out_slice(plane, 0), sout0).wait()
        unscatter(0)

    pltpu.make_async_copy(ov1, out_slice(base + _PW - 1, 1), sout1).wait()


_unpool = functools.partial(
    pl.kernel,
    out_type=jax.ShapeDtypeStruct((_N, _HO, _WO), jnp.float32),
    mesh=plsc.VectorSubcoreMesh(
        core_axis_name="c", subcore_axis_name="s",
        num_cores=_NC, num_subcores=_NS,
    ),
    scratch_types=[
        pltpu.VMEM((_HH, _W), jnp.float32),
        pltpu.VMEM((_HH, _W), jnp.float32),
        pltpu.VMEM((_HH, _W), jnp.int32),
        pltpu.VMEM((_HH, _W), jnp.int32),
        pltpu.VMEM((_HH, _W), jnp.int32),
        pltpu.VMEM((_HH, _W), jnp.int32),
        pltpu.VMEM((_HOH, _WO), jnp.float32),
        pltpu.VMEM((_HOH, _WO), jnp.float32),
        pltpu.SemaphoreType.DMA,
        pltpu.SemaphoreType.DMA,
        pltpu.SemaphoreType.DMA,
        pltpu.SemaphoreType.DMA,
    ],
    compiler_params=pltpu.CompilerParams(
        needs_layout_passes=False, use_tc_tiling_on_sc=True,
    ),
)(_unpool_body)


@jax.jit
def kernel(x, indices):
    out = _unpool(x.reshape(_N, _H, _W), indices.reshape(_N, _H, _W))
    return out.reshape(_B, _C, _HO, _WO)
